# baseline (speedup 1.0000x reference)
import functools

import jax
import jax.numpy as jnp
from jax import lax
from jax.experimental import pallas as pl
from jax.experimental.pallas import tpu as pltpu
from jax.experimental.pallas import tpu_sc as plsc

NC = 2
NS = 16
NW = NC * NS
LANES = 16
GRP = 128
CHUNK = 64


def _splat_lane(vec, lane):
    idx = jnp.full((LANES, 1), lane, jnp.int32)
    dn = lax.GatherDimensionNumbers(
        offset_dims=(), collapsed_slice_dims=(0,), start_index_map=(0,))
    return lax.gather(vec, idx, dn, (1,),
                      mode=lax.GatherScatterMode.PROMISE_IN_BOUNDS)


def _sc_body(seq, nch, tids_hbm, ids_hbm, ttab_hbm, xtab_hbm,
             out_hbm, rib, vidx, g3, acc, tidx, tbuf, obuf, row0, sem):
    ngrp = (CHUNK * seq) // GRP
    wid = lax.axis_index("s") * NC + lax.axis_index("c")

    pltpu.sync_copy(xtab_hbm.at[pl.ds(0, 1)], row0)

    def chunk_body(c, _):
        g = wid * nch + c
        pltpu.sync_copy(ids_hbm.at[g], rib)
        pltpu.sync_copy(tids_hbm.at[g], tidx)

        def remap_body(jr, _):
            for k in range(GRP // LANES):
                i_vec = (jnp.arange(LANES, dtype=jnp.int32)
                         + (jr * GRP + k * LANES))
                v = rib[jr, pl.ds(k * LANES, LANES)]
                s = lax.shift_right_logical(i_vec * 1311, 16)
                t = i_vec * CHUNK - s * (CHUNK * seq - 1)
                r_vec = lax.shift_right_logical(t, 7)
                c_vec = jnp.bitwise_and(t, GRP - 1)
                plsc.store_scatter(vidx, [r_vec, c_vec], v)
            return _
        lax.fori_loop(0, ngrp, remap_body, None)

        zf = jnp.zeros((LANES,), jnp.float32)
        def zero_body(b, _):
            acc[b, pl.ds(0, LANES)] = zf
            acc[b, pl.ds(LANES, LANES)] = zf
            return _
        lax.fori_loop(0, CHUNK, zero_body, None)

        descs = [pltpu.async_copy(xtab_hbm.at[vidx.at[j]], g3.at[j], sem)
                 for j in range(ngrp)]
        descs.append(pltpu.async_copy(ttab_hbm.at[tidx], tbuf, sem))
        for dsc in descs:
            dsc.wait()

        def acc_body(r, zcnt):
            zout = list(zcnt)
            for h in range(2):
                for k in range(CHUNK // LANES):
                    col = CHUNK * h + k * LANES
                    jv = vidx[r, pl.ds(col, LANES)]
                    zout[k] = zout[k] + jnp.where(jv == 0, 1.0, 0.0)
                    for m in range(LANES):
                        b = k * LANES + m
                        row = col + m
                        plsc.addupdate(acc.at[b, pl.ds(0, LANES)],
                                       g3[r, row, pl.ds(0, LANES)])
                        plsc.addupdate(acc.at[b, pl.ds(LANES, LANES)],
                                       g3[r, row, pl.ds(LANES, LANES)])
            return tuple(zout)

        zcnt = lax.fori_loop(0, ngrp, acc_body,
                             (zf,) * (CHUNK // LANES))

        r0a = row0[0, pl.ds(0, LANES)]
        r0b = row0[0, pl.ds(LANES, LANES)]
        for k in range(CHUNK // LANES):
            z = zcnt[k]
            inv = 1.0 / jnp.maximum(float(seq) - z, 1.0)
            for m in range(LANES):
                b = k * LANES + m
                zm = _splat_lane(z, m)
                ivm = _splat_lane(inv, m)
                o = b * 64
                obuf[pl.ds(o, LANES)] = tbuf[b, pl.ds(0, LANES)]
                obuf[pl.ds(o + 16, LANES)] = tbuf[b, pl.ds(LANES, LANES)]
                obuf[pl.ds(o + 32, LANES)] = (
                    (acc[b, pl.ds(0, LANES)] - zm * r0a) * ivm)
                obuf[pl.ds(o + 48, LANES)] = (
                    (acc[b, pl.ds(LANES, LANES)] - zm * r0b) * ivm)

        pltpu.sync_copy(obuf, out_hbm.at[g])
        return _
    lax.fori_loop(0, nch, chunk_body, None)


def kernel(title_ids, text_ids, title_table, text_table):
    b, seq = text_ids.shape
    d = title_table.shape[1]
    assert d == 32 and b % NW == 0 and (CHUNK * seq) % GRP == 0
    nch = b // (NW * CHUNK)
    ngrp = (CHUNK * seq) // GRP
    nglobal = b // CHUNK

    ids3 = text_ids.astype(jnp.int32).reshape(nglobal, ngrp, GRP)
    tids2 = title_ids.astype(jnp.int32).reshape(nglobal, CHUNK)

    mesh = plsc.VectorSubcoreMesh(core_axis_name="c", subcore_axis_name="s")
    run = pl.kernel(
        functools.partial(_sc_body, seq, nch),
        out_type=jax.ShapeDtypeStruct((nglobal, CHUNK * 2 * d), jnp.float32),
        mesh=mesh,
        scratch_types=[
            pltpu.VMEM((ngrp, GRP), jnp.int32),
            pltpu.VMEM((ngrp, GRP), jnp.int32),
            pltpu.VMEM((ngrp, GRP, d), jnp.float32),
            pltpu.VMEM((CHUNK, d), jnp.float32),
            pltpu.VMEM((CHUNK,), jnp.int32),
            pltpu.VMEM((CHUNK, d), jnp.float32),
            pltpu.VMEM((CHUNK * 2 * d,), jnp.float32),
            pltpu.VMEM((1, d), jnp.float32),
            pltpu.SemaphoreType.DMA,
        ],
        compiler_params=pltpu.CompilerParams(
            use_tc_tiling_on_sc=False, needs_layout_passes=False),
    )
    out = run(tids2, ids3, title_table, text_table)
    return out.reshape(b, 2 * d)
